# baseline (speedup 1.0000x reference)
import jax
import jax.numpy as jnp
from jax.experimental import pallas as pl
from jax.experimental.pallas import tpu as pltpu

K = 100000
N = 16

BC = 2048
NBUF = 4
NCF = K // BC
TAIL = K - NCF * BC


def _mm_kernel(x_hbm, emb_ref, embt_ref, o_ref, buf, tbuf, sems, tsem):
    def start_copy(c):
        slot = jax.lax.rem(c, NBUF)
        pltpu.make_async_copy(
            x_hbm.at[:, pl.ds(c * BC, BC)],
            buf.at[slot],
            sems.at[slot],
        ).start()

    for c in range(NBUF):
        start_copy(c)
    tail_copy = pltpu.make_async_copy(
        x_hbm.at[:, pl.ds(NCF * BC, TAIL)], tbuf, tsem)
    tail_copy.start()

    o_ref[...] = jnp.zeros_like(o_ref)

    def body(c, carry):
        slot = jax.lax.rem(c, NBUF)
        pltpu.make_async_copy(
            x_hbm.at[:, pl.ds(c * BC, BC)],
            buf.at[slot],
            sems.at[slot],
        ).wait()
        et = emb_ref[:, pl.ds(c * BC, BC)]
        o_ref[...] += jax.lax.dot_general(
            buf[slot], et, (((1,), (1,)), ((), ())),
            preferred_element_type=jnp.float32)

        @pl.when(c + NBUF < NCF)
        def _():
            start_copy(c + NBUF)

        return carry

    jax.lax.fori_loop(0, NCF, body, 0, unroll=False)

    tail_copy.wait()
    o_ref[...] += jax.lax.dot_general(
        tbuf[...], embt_ref[...], (((1,), (1,)), ((), ())),
        preferred_element_type=jnp.float32)


def kernel(inputs, embeddings):
    m = inputs.shape[0]
    emb_t = embeddings.T
    emb_tail = emb_t[:, NCF * BC:]

    return pl.pallas_call(
        _mm_kernel,
        in_specs=[
            pl.BlockSpec(memory_space=pl.ANY),
            pl.BlockSpec(memory_space=pltpu.MemorySpace.VMEM),
            pl.BlockSpec(memory_space=pltpu.MemorySpace.VMEM),
        ],
        out_specs=pl.BlockSpec(memory_space=pltpu.MemorySpace.VMEM),
        out_shape=jax.ShapeDtypeStruct((m, N), jnp.float32),
        scratch_shapes=[
            pltpu.MemorySpace.VMEM((NBUF, m, BC), jnp.float32),
            pltpu.MemorySpace.VMEM((m, TAIL), jnp.float32),
            pltpu.SemaphoreType.DMA((NBUF,)),
            pltpu.SemaphoreType.DMA,
        ],
    )(inputs, emb_t, emb_tail)


# speedup vs baseline: 1.0078x; 1.0078x over previous
import jax
import jax.numpy as jnp
from jax.experimental import pallas as pl
from jax.experimental.pallas import tpu as pltpu

K = 100000
N = 16

RB = 32
NBUF = 3


def _mm_kernel(x_hbm, emb_ref, o_ref, buf, sems):
    m = o_ref.shape[0]
    nc = m // RB
    nbuf = min(NBUF, nc)

    def start_copy(c):
        slot = jax.lax.rem(c, nbuf)
        pltpu.make_async_copy(
            x_hbm.at[pl.ds(c * RB, RB), :],
            buf.at[slot],
            sems.at[slot],
        ).start()

    for c in range(nbuf):
        start_copy(c)

    def body(c, carry):
        slot = jax.lax.rem(c, nbuf)
        pltpu.make_async_copy(
            x_hbm.at[pl.ds(c * RB, RB), :],
            buf.at[slot],
            sems.at[slot],
        ).wait()
        o_ref[pl.ds(c * RB, RB), :] = jax.lax.dot_general(
            buf[slot], emb_ref[...], (((1,), (1,)), ((), ())),
            preferred_element_type=jnp.float32)

        @pl.when(c + nbuf < nc)
        def _():
            start_copy(c + nbuf)

        return carry

    jax.lax.fori_loop(0, nc, body, 0, unroll=False)


def kernel(inputs, embeddings):
    m = inputs.shape[0]
    emb_t = embeddings.T

    return pl.pallas_call(
        _mm_kernel,
        in_specs=[
            pl.BlockSpec(memory_space=pl.ANY),
            pl.BlockSpec(memory_space=pltpu.MemorySpace.VMEM),
        ],
        out_specs=pl.BlockSpec(memory_space=pltpu.MemorySpace.VMEM),
        out_shape=jax.ShapeDtypeStruct((m, N), jnp.float32),
        scratch_shapes=[
            pltpu.MemorySpace.VMEM((min(NBUF, m // RB), RB, K), jnp.float32),
            pltpu.SemaphoreType.DMA((min(NBUF, m // RB),)),
        ],
    )(inputs, emb_t)
